# baseline (speedup 1.0000x reference)
import functools

import numpy as np
import jax
import jax.numpy as jnp
from jax import lax
from jax.experimental import pallas as pl
from jax.experimental.pallas import tpu as pltpu
from jax.experimental.pallas import tpu_sc as plsc

B = 16384
H = 200
E = 64
HID = 256
VOCABN = 1000000

NW = 32
BPW = B // NW
CB = 8
NIDX = CB * H
NCH = BPW // CB
UNROLL = 8
FLAT_WINDOW = 32

_PERM = np.concatenate(
    [np.concatenate([32 * c + np.arange(0, 32, 2), 32 * c + np.arange(1, 32, 2)])
     for c in range(E // 32)]
)


HA = 128
HB = H - HA


def _sc_pool(idxa_hbm, idxb_hbm, table_hbm, out_hbm,
             ia0, ia1, ia2, ia3, ib0, ib1, ib2, ib3, r0, r1, s0, s1,
             is0, is1, is2, is3, gs0, gs1, osem):
    wid = lax.axis_index("s") * 2 + lax.axis_index("c")
    base_row = wid * BPW

    idxas = (ia0, ia1, ia2, ia3)
    idxbs = (ib0, ib1, ib2, ib3)
    rows = (r0, r1)
    stages = (s0, s1)
    isems = (is0, is1, is2, is3)
    gsems = (gs0, gs1)

    def idx_fetch(chunk, j):
        start = pl.multiple_of(base_row + chunk * CB, CB)
        pltpu.async_copy(idxa_hbm.at[pl.ds(start, CB), :], idxas[j],
                         isems[j])
        pltpu.async_copy(idxb_hbm.at[pl.ds(start, CB), :], idxbs[j],
                         isems[j])

    def gather_issue(j, rb):
        pltpu.make_async_copy(idxa_hbm.at[pl.ds(0, CB), :], idxas[j],
                              isems[j]).wait()
        pltpu.make_async_copy(idxb_hbm.at[pl.ds(0, CB), :], idxbs[j],
                              isems[j]).wait()
        for r in range(CB):
            pltpu.async_copy(table_hbm.at[idxas[j].at[r]],
                             rows[rb].at[r, pl.ds(0, HA)], gsems[rb])
            pltpu.async_copy(table_hbm.at[idxbs[j].at[r]],
                             rows[rb].at[r, pl.ds(HA, HB)], gsems[rb])

    for c in range(4):
        idx_fetch(c, c)
    gather_issue(0, 0)
    gather_issue(1, 1)

    def outer(g, _):
        for b4 in range(4):
            t = g * 4 + b4
            rb = b4 % 2
            for r in range(CB):
                pltpu.make_async_copy(
                    table_hbm.at[idxas[b4].at[r]],
                    rows[rb].at[r, pl.ds(0, HA)], gsems[rb]).wait()
                pltpu.make_async_copy(
                    table_hbm.at[idxbs[b4].at[r]],
                    rows[rb].at[r, pl.ds(HA, HB)], gsems[rb]).wait()
            @pl.when(t >= 2)
            def _(rb=rb):
                pltpu.make_async_copy(stages[rb],
                                      out_hbm.at[pl.ds(0, CB)], osem).wait()
            for r in range(CB):
                def jbody(jj, accs, r=r, rb=rb):
                    accs = list(accs)
                    for u in range(UNROLL):
                        row = jj * UNROLL + u
                        for c in range(E // 32):
                            w = rows[rb][r, row, pl.ds(c * 16, 16)]
                            ab = plsc.bitcast(w, jnp.bfloat16)
                            lo, hi = plsc.unpack(
                                ab, format=plsc.PackFormat.INTERLEAVED)
                            accs[2 * c] = accs[2 * c] + lo
                            accs[2 * c + 1] = accs[2 * c + 1] + hi
                    return tuple(accs)

                zero = jnp.zeros((16,), jnp.float32)
                accs = lax.fori_loop(0, H // UNROLL, jbody,
                                     (zero,) * (E // 16))
                for c in range(E // 16):
                    stages[rb][r, pl.ds(c * 16, 16)] = accs[c]
            out_start = pl.multiple_of(base_row + t * CB, CB)
            pltpu.async_copy(stages[rb], out_hbm.at[pl.ds(out_start, CB)],
                             osem)
            @pl.when(t + 4 < NCH)
            def _(t=t, b4=b4):
                idx_fetch(t + 4, b4)
            @pl.when(t + 2 < NCH)
            def _(b4=b4, rb=rb):
                gather_issue((b4 + 2) % 4, rb)
        return _

    lax.fori_loop(0, NCH // 4, outer, None)
    for _ in range(2):
        pltpu.make_async_copy(stages[0], out_hbm.at[pl.ds(0, CB)],
                              osem).wait()


def _mlp_body(x_ref, w1_ref, b1_ref, w2_ref, b2_ref, o_ref):
    x = x_ref[...] * (1.0 / H)
    h = jnp.dot(x, w1_ref[...], preferred_element_type=jnp.float32)
    h = jnp.maximum(h + b1_ref[...], 0.0)
    z = jnp.sum(h * w2_ref[...], axis=1, keepdims=True) + b2_ref[...]
    o_ref[...] = 1.0 / (1.0 + jnp.exp(-z))


def kernel(inputs, table, W1, b1, W2, b2):
    idx32 = inputs.astype(jnp.int32)
    idxa = idx32[:, :HA]
    idxb = idx32[:, HA:]
    table_bf = table.astype(jnp.bfloat16)
    table_pk = jax.lax.bitcast_convert_type(
        table_bf.reshape(VOCABN, E // 2, 2), jnp.int32)
    W1p = W1[_PERM, :]

    mesh = plsc.VectorSubcoreMesh(core_axis_name="c", subcore_axis_name="s")
    pooled = pl.kernel(
        _sc_pool,
        out_type=jax.ShapeDtypeStruct((B, E), jnp.float32),
        mesh=mesh,
        compiler_params=pltpu.CompilerParams(
            use_tc_tiling_on_sc=False, needs_layout_passes=False),
        scratch_types=[
            pltpu.VMEM((CB, HA), jnp.int32),
            pltpu.VMEM((CB, HA), jnp.int32),
            pltpu.VMEM((CB, HA), jnp.int32),
            pltpu.VMEM((CB, HA), jnp.int32),
            pltpu.VMEM((CB, HB), jnp.int32),
            pltpu.VMEM((CB, HB), jnp.int32),
            pltpu.VMEM((CB, HB), jnp.int32),
            pltpu.VMEM((CB, HB), jnp.int32),
            pltpu.VMEM((CB, H, E // 2), jnp.int32),
            pltpu.VMEM((CB, H, E // 2), jnp.int32),
            pltpu.VMEM((CB, E), jnp.float32),
            pltpu.VMEM((CB, E), jnp.float32),
            pltpu.SemaphoreType.DMA,
            pltpu.SemaphoreType.DMA,
            pltpu.SemaphoreType.DMA,
            pltpu.SemaphoreType.DMA,
            pltpu.SemaphoreType.DMA,
            pltpu.SemaphoreType.DMA,
            pltpu.SemaphoreType.DMA,
        ],
    )(idxa, idxb, table_pk)

    BM = 2048
    out = pl.pallas_call(
        _mlp_body,
        grid=(B // BM,),
        in_specs=[
            pl.BlockSpec((BM, E), lambda i: (i, 0)),
            pl.BlockSpec((E, HID), lambda i: (0, 0)),
            pl.BlockSpec((1, HID), lambda i: (0, 0)),
            pl.BlockSpec((1, HID), lambda i: (0, 0)),
            pl.BlockSpec((1, 1), lambda i: (0, 0)),
        ],
        out_specs=pl.BlockSpec((BM, 1), lambda i: (i, 0)),
        out_shape=jax.ShapeDtypeStruct((B, 1), jnp.float32),
    )(pooled, W1p, b1.reshape(1, HID), W2.reshape(1, HID), b2.reshape(1, 1))
    return out


# speedup vs baseline: 1.8307x; 1.8307x over previous
import functools

import numpy as np
import jax
import jax.numpy as jnp
from jax import lax
from jax.experimental import pallas as pl
from jax.experimental.pallas import tpu as pltpu
from jax.experimental.pallas import tpu_sc as plsc

B = 16384
H = 200
E = 64
HID = 256
VOCABN = 1000000

NW = 32
BPW = B // NW
CB = 8
NIDX = CB * H
NCH = BPW // CB
UNROLL = 8
FLAT_WINDOW = 32

_PERM = np.concatenate(
    [np.concatenate([32 * c + np.arange(0, 32, 2), 32 * c + np.arange(1, 32, 2)])
     for c in range(E // 32)]
)


def _sc_pool(idx_hbm, table_hbm, out_hbm,
             i0, i1, i2, i3, r0, r1, s0, s1,
             is0, is1, is2, is3, gs0, gs1, osem):
    wid = lax.axis_index("s") * 2 + lax.axis_index("c")
    base_row = wid * BPW

    idxs = (i0, i1, i2, i3)
    rows = (r0, r1)
    stages = (s0, s1)
    isems = (is0, is1, is2, is3)
    gsems = (gs0, gs1)

    def idx_fetch(chunk, j):
        start = pl.multiple_of((base_row + chunk * CB) * H, NIDX)
        pltpu.async_copy(idx_hbm.at[pl.ds(start, NIDX)], idxs[j], isems[j])

    def gather_issue(j, rb):
        pltpu.make_async_copy(idx_hbm.at[pl.ds(0, NIDX)], idxs[j],
                              isems[j]).wait()
        for r in range(CB):
            pltpu.async_copy(table_hbm.at[idxs[j].at[pl.ds(r * H, H)]],
                             rows[rb].at[r], gsems[rb])

    for c in range(4):
        idx_fetch(c, c)
    gather_issue(0, 0)
    gather_issue(1, 1)

    def outer(g, _):
        for b4 in range(4):
            t = g * 4 + b4
            rb = b4 % 2
            for r in range(CB):
                pltpu.make_async_copy(
                    table_hbm.at[idxs[b4].at[pl.ds(r * H, H)]],
                    rows[rb].at[r], gsems[rb]).wait()
            @pl.when(t >= 2)
            def _(rb=rb):
                pltpu.make_async_copy(stages[rb],
                                      out_hbm.at[pl.ds(0, CB)], osem).wait()
            for r in range(CB):
                def jbody(jj, accs, r=r, rb=rb):
                    accs = list(accs)
                    for u in range(UNROLL):
                        row = jj * UNROLL + u
                        for c in range(E // 32):
                            ab = rows[rb][r, row, pl.ds(c * 32, 32)]
                            lo, hi = plsc.unpack(
                                ab, format=plsc.PackFormat.INTERLEAVED)
                            accs[2 * c] = accs[2 * c] + lo
                            accs[2 * c + 1] = accs[2 * c + 1] + hi
                    return tuple(accs)

                zero = jnp.zeros((16,), jnp.float32)
                accs = lax.fori_loop(0, H // UNROLL, jbody,
                                     (zero,) * (E // 16))
                for c in range(E // 16):
                    stages[rb][r, pl.ds(c * 16, 16)] = accs[c]
            out_start = pl.multiple_of(base_row + t * CB, CB)
            pltpu.async_copy(stages[rb], out_hbm.at[pl.ds(out_start, CB)],
                             osem)
            @pl.when(t + 4 < NCH)
            def _(t=t, b4=b4):
                idx_fetch(t + 4, b4)
            @pl.when(t + 2 < NCH)
            def _(b4=b4, rb=rb):
                gather_issue((b4 + 2) % 4, rb)
        return _

    lax.fori_loop(0, NCH // 4, outer, None)
    for _ in range(2):
        pltpu.make_async_copy(stages[0], out_hbm.at[pl.ds(0, CB)],
                              osem).wait()


def _mlp_body(x_ref, w1_ref, b1_ref, w2_ref, b2_ref, o_ref):
    x = x_ref[...] * (1.0 / H)
    h = jnp.dot(x, w1_ref[...], preferred_element_type=jnp.float32)
    h = jnp.maximum(h + b1_ref[...], 0.0)
    z = jnp.sum(h * w2_ref[...], axis=1, keepdims=True) + b2_ref[...]
    o_ref[...] = 1.0 / (1.0 + jnp.exp(-z))


def kernel(inputs, table, W1, b1, W2, b2):
    idx_flat = inputs.reshape(-1).astype(jnp.int32)
    table_bf = table.astype(jnp.bfloat16)
    W1p = W1[_PERM, :]

    mesh = plsc.VectorSubcoreMesh(core_axis_name="c", subcore_axis_name="s")
    pooled = pl.kernel(
        _sc_pool,
        out_type=jax.ShapeDtypeStruct((B, E), jnp.float32),
        mesh=mesh,
        compiler_params=pltpu.CompilerParams(
            use_tc_tiling_on_sc=False, needs_layout_passes=False),
        scratch_types=[
            pltpu.VMEM((NIDX,), jnp.int32),
            pltpu.VMEM((NIDX,), jnp.int32),
            pltpu.VMEM((NIDX,), jnp.int32),
            pltpu.VMEM((NIDX,), jnp.int32),
            pltpu.VMEM((CB, H, E), jnp.bfloat16),
            pltpu.VMEM((CB, H, E), jnp.bfloat16),
            pltpu.VMEM((CB, E), jnp.float32),
            pltpu.VMEM((CB, E), jnp.float32),
            pltpu.SemaphoreType.DMA,
            pltpu.SemaphoreType.DMA,
            pltpu.SemaphoreType.DMA,
            pltpu.SemaphoreType.DMA,
            pltpu.SemaphoreType.DMA,
            pltpu.SemaphoreType.DMA,
            pltpu.SemaphoreType.DMA,
        ],
    )(idx_flat, table_bf)

    BM = 2048
    out = pl.pallas_call(
        _mlp_body,
        grid=(B // BM,),
        in_specs=[
            pl.BlockSpec((BM, E), lambda i: (i, 0)),
            pl.BlockSpec((E, HID), lambda i: (0, 0)),
            pl.BlockSpec((1, HID), lambda i: (0, 0)),
            pl.BlockSpec((1, HID), lambda i: (0, 0)),
            pl.BlockSpec((1, 1), lambda i: (0, 0)),
        ],
        out_specs=pl.BlockSpec((BM, 1), lambda i: (i, 0)),
        out_shape=jax.ShapeDtypeStruct((B, 1), jnp.float32),
    )(pooled, W1p, b1.reshape(1, HID), W2.reshape(1, HID), b2.reshape(1, 1))
    return out


# speedup vs baseline: 2.0121x; 1.0991x over previous
import functools

import numpy as np
import jax
import jax.numpy as jnp
from jax import lax
from jax.experimental import pallas as pl
from jax.experimental.pallas import tpu as pltpu
from jax.experimental.pallas import tpu_sc as plsc

B = 16384
H = 200
E = 64
HID = 256
VOCABN = 1000000

NW = 32
BPW = B // NW
CB = 4
NIDX = CB * H
NCH = BPW // CB
UNROLL = 8
FLAT_WINDOW = 32

_PERM = np.concatenate(
    [np.concatenate([32 * c + np.arange(0, 32, 2), 32 * c + np.arange(1, 32, 2)])
     for c in range(E // 32)]
)


def _sc_pool(idx_hbm, table_hbm, out_hbm,
             i0, i1, i2, i3, r0, r1, s0, s1,
             is0, is1, is2, is3, gs0, gs1, osem):
    wid = lax.axis_index("s") * 2 + lax.axis_index("c")
    base_row = wid * BPW

    idxs = (i0, i1, i2, i3)
    rows = (r0, r1)
    stages = (s0, s1)
    isems = (is0, is1, is2, is3)
    gsems = (gs0, gs1)

    def idx_fetch(chunk, j):
        start = pl.multiple_of((base_row + chunk * CB) * H, NIDX)
        pltpu.async_copy(idx_hbm.at[pl.ds(start, NIDX)], idxs[j], isems[j])

    def gather_issue(j, rb):
        pltpu.make_async_copy(idx_hbm.at[pl.ds(0, NIDX)], idxs[j],
                              isems[j]).wait()
        for r in range(CB):
            pltpu.async_copy(table_hbm.at[idxs[j].at[pl.ds(r * H, H)]],
                             rows[rb].at[r], gsems[rb])

    for c in range(4):
        idx_fetch(c, c)
    gather_issue(0, 0)
    gather_issue(1, 1)

    def outer(g, _):
        for b4 in range(4):
            t = g * 4 + b4
            rb = b4 % 2
            for r in range(CB):
                pltpu.make_async_copy(
                    table_hbm.at[idxs[b4].at[pl.ds(r * H, H)]],
                    rows[rb].at[r], gsems[rb]).wait()
            @pl.when(t >= 2)
            def _(rb=rb):
                pltpu.make_async_copy(stages[rb],
                                      out_hbm.at[pl.ds(0, CB)], osem).wait()
            for r in range(CB):
                def jbody(jj, accs, r=r, rb=rb):
                    accs = list(accs)
                    for u in range(UNROLL):
                        row = jj * UNROLL + u
                        for c in range(E // 16):
                            accs[c] = accs[c] + rows[rb][r, row,
                                                         pl.ds(c * 16, 16)]
                    return tuple(accs)

                zero = jnp.zeros((16,), jnp.float32)
                accs = lax.fori_loop(0, H // UNROLL, jbody,
                                     (zero,) * (E // 16))
                for c in range(E // 16):
                    stages[rb][r, pl.ds(c * 16, 16)] = accs[c]
            out_start = pl.multiple_of(base_row + t * CB, CB)
            pltpu.async_copy(stages[rb], out_hbm.at[pl.ds(out_start, CB)],
                             osem)
            @pl.when(t + 4 < NCH)
            def _(t=t, b4=b4):
                idx_fetch(t + 4, b4)
            @pl.when(t + 2 < NCH)
            def _(b4=b4, rb=rb):
                gather_issue((b4 + 2) % 4, rb)
        return _

    lax.fori_loop(0, NCH // 4, outer, None)
    for _ in range(2):
        pltpu.make_async_copy(stages[0], out_hbm.at[pl.ds(0, CB)],
                              osem).wait()


def _mlp_body(x_ref, w1_ref, b1_ref, w2_ref, b2_ref, o_ref):
    x = x_ref[...] * (1.0 / H)
    h = jnp.dot(x, w1_ref[...], preferred_element_type=jnp.float32)
    h = jnp.maximum(h + b1_ref[...], 0.0)
    z = jnp.sum(h * w2_ref[...], axis=1, keepdims=True) + b2_ref[...]
    o_ref[...] = 1.0 / (1.0 + jnp.exp(-z))


def kernel(inputs, table, W1, b1, W2, b2):
    idx_flat = inputs.reshape(-1).astype(jnp.int32)

    mesh = plsc.VectorSubcoreMesh(core_axis_name="c", subcore_axis_name="s")
    pooled = pl.kernel(
        _sc_pool,
        out_type=jax.ShapeDtypeStruct((B, E), jnp.float32),
        mesh=mesh,
        compiler_params=pltpu.CompilerParams(
            use_tc_tiling_on_sc=False, needs_layout_passes=False),
        scratch_types=[
            pltpu.VMEM((NIDX,), jnp.int32),
            pltpu.VMEM((NIDX,), jnp.int32),
            pltpu.VMEM((NIDX,), jnp.int32),
            pltpu.VMEM((NIDX,), jnp.int32),
            pltpu.VMEM((CB, H, E), jnp.float32),
            pltpu.VMEM((CB, H, E), jnp.float32),
            pltpu.VMEM((CB, E), jnp.float32),
            pltpu.VMEM((CB, E), jnp.float32),
            pltpu.SemaphoreType.DMA,
            pltpu.SemaphoreType.DMA,
            pltpu.SemaphoreType.DMA,
            pltpu.SemaphoreType.DMA,
            pltpu.SemaphoreType.DMA,
            pltpu.SemaphoreType.DMA,
            pltpu.SemaphoreType.DMA,
        ],
    )(idx_flat, table)

    BM = 2048
    out = pl.pallas_call(
        _mlp_body,
        grid=(B // BM,),
        in_specs=[
            pl.BlockSpec((BM, E), lambda i: (i, 0)),
            pl.BlockSpec((E, HID), lambda i: (0, 0)),
            pl.BlockSpec((1, HID), lambda i: (0, 0)),
            pl.BlockSpec((1, HID), lambda i: (0, 0)),
            pl.BlockSpec((1, 1), lambda i: (0, 0)),
        ],
        out_specs=pl.BlockSpec((BM, 1), lambda i: (i, 0)),
        out_shape=jax.ShapeDtypeStruct((B, 1), jnp.float32),
    )(pooled, W1, b1.reshape(1, HID), W2.reshape(1, HID), b2.reshape(1, 1))
    return out
